# baseline (speedup 1.0000x reference)
import jax
import jax.numpy as jnp
from jax.experimental import pallas as pl
from jax.experimental.pallas import tpu as pltpu

NUM_EXPERTS = 8
RANK = 4
D_MODEL = 768
D_FF = 3072
TOP_K = 2
FF_TILE = 512
N_FF = D_FF // FF_TILE

_CONTRACT_LAST = (((1,), (1,)), ((), ()))


def _moe_kernel(idx_ref, val_ref, x_ref, wi_ref, wo_ref,
                a0_ref, a1_ref, b0_ref, b1_ref, out_ref,
                p0_scr, p1_scr, x16_scr, s16_scr):
    ff = pl.program_id(0)

    @pl.when(ff == 0)
    def _():
        xs = (val_ref[0] + val_ref[1]) * x_ref[...]
        p0_scr[...] = jax.lax.dot_general(xs, a0_ref[0], _CONTRACT_LAST,
                                          preferred_element_type=jnp.float32
                                          ).astype(jnp.bfloat16)
        p1_scr[...] = jax.lax.dot_general(xs, a1_ref[0], _CONTRACT_LAST,
                                          preferred_element_type=jnp.float32
                                          ).astype(jnp.bfloat16)
        x16_scr[...] = xs.astype(jnp.bfloat16)

    for c in range(2):
        sl = slice(c * (FF_TILE // 2), (c + 1) * (FF_TILE // 2))
        base = jax.lax.dot_general(x16_scr[...],
                                   wi_ref[sl, :].astype(jnp.bfloat16),
                                   _CONTRACT_LAST,
                                   preferred_element_type=jnp.float32)
        l0 = jax.lax.dot_general(p0_scr[...], b0_ref[0, sl].astype(jnp.bfloat16),
                                 _CONTRACT_LAST,
                                 preferred_element_type=jnp.float32)
        l1 = jax.lax.dot_general(p1_scr[...], b1_ref[0, sl].astype(jnp.bfloat16),
                                 _CONTRACT_LAST,
                                 preferred_element_type=jnp.float32)
        s = jnp.maximum(base + l0, 0.0) + jnp.maximum(base + l1, 0.0)
        s16_scr[:, sl] = s.astype(jnp.bfloat16)
    s16 = s16_scr[...]

    third = D_MODEL // 2
    for q in range(2):
        lo, hi = q * third, (q + 1) * third
        c_q = jax.lax.dot_general(s16, wo_ref[lo:hi, :].astype(jnp.bfloat16),
                                  _CONTRACT_LAST,
                                  preferred_element_type=jnp.float32)
        out_ref[:, lo:hi] = jnp.where(ff == 0, c_q, out_ref[:, lo:hi] + c_q)


def kernel(hidden_states, top_k_indices, top_k_values, Wi, Wo, lora_As, lora_Bs):
    batch, seq, d_model = hidden_states.shape
    rows = batch * seq
    x = hidden_states.reshape(rows, d_model)
    idx = top_k_indices.astype(jnp.int32)
    vals = top_k_values.astype(jnp.float32)

    out = pl.pallas_call(
        _moe_kernel,
        grid_spec=pltpu.PrefetchScalarGridSpec(
            num_scalar_prefetch=2,
            grid=(N_FF,),
            in_specs=[
                pl.BlockSpec((rows, d_model), lambda ff, idx, val: (0, 0)),
                pl.BlockSpec((FF_TILE, d_model), lambda ff, idx, val: (ff, 0)),
                pl.BlockSpec((d_model, FF_TILE), lambda ff, idx, val: (0, ff)),
                pl.BlockSpec((1, RANK, d_model), lambda ff, idx, val: (idx[0], 0, 0)),
                pl.BlockSpec((1, RANK, d_model), lambda ff, idx, val: (idx[1], 0, 0)),
                pl.BlockSpec((1, FF_TILE, RANK), lambda ff, idx, val: (idx[0], ff, 0)),
                pl.BlockSpec((1, FF_TILE, RANK), lambda ff, idx, val: (idx[1], ff, 0)),
            ],
            out_specs=pl.BlockSpec((rows, d_model), lambda ff, idx, val: (0, 0)),
            scratch_shapes=[
                pltpu.VMEM((rows, RANK), jnp.bfloat16),
                pltpu.VMEM((rows, RANK), jnp.bfloat16),
                pltpu.VMEM((rows, d_model), jnp.bfloat16),
                pltpu.VMEM((rows, FF_TILE), jnp.bfloat16),
            ],
        ),
        out_shape=jax.ShapeDtypeStruct((rows, d_model), jnp.float32),
    )(idx, vals, x, Wi, Wo, lora_As, lora_As, lora_Bs, lora_Bs)
    return out.reshape(batch, seq, d_model)


# speedup vs baseline: 1.0704x; 1.0704x over previous
import jax
import jax.numpy as jnp
from jax.experimental import pallas as pl
from jax.experimental.pallas import tpu as pltpu

NUM_EXPERTS = 8
RANK = 4
D_MODEL = 768
D_FF = 3072
TOP_K = 2
FF_TILE = 512
N_FF = D_FF // FF_TILE

_CONTRACT_LAST = (((1,), (1,)), ((), ()))


def _moe_kernel(idx_ref, val_ref, x_ref, wi_ref, wo_ref,
                a0_ref, a1_ref, b0_ref, b1_ref, out_ref,
                p0_scr, p1_scr, x16_scr, s16_scr):
    ff = pl.program_id(0)

    @pl.when(ff == 0)
    def _():
        xs = (val_ref[0] + val_ref[1]) * x_ref[...]
        p0_scr[...] = jax.lax.dot_general(xs, a0_ref[0], _CONTRACT_LAST,
                                          preferred_element_type=jnp.float32
                                          ).astype(jnp.bfloat16)
        p1_scr[...] = jax.lax.dot_general(xs, a1_ref[0], _CONTRACT_LAST,
                                          preferred_element_type=jnp.float32
                                          ).astype(jnp.bfloat16)
        x16_scr[...] = xs.astype(jnp.bfloat16)

    for c in range(2):
        sl = slice(c * (FF_TILE // 2), (c + 1) * (FF_TILE // 2))
        base = jax.lax.dot_general(x16_scr[...],
                                   wi_ref[sl, :].astype(jnp.bfloat16),
                                   _CONTRACT_LAST,
                                   preferred_element_type=jnp.float32)
        l0 = jax.lax.dot_general(p0_scr[...], b0_ref[0, sl].astype(jnp.bfloat16),
                                 _CONTRACT_LAST,
                                 preferred_element_type=jnp.float32)
        l1 = jax.lax.dot_general(p1_scr[...], b1_ref[0, sl].astype(jnp.bfloat16),
                                 _CONTRACT_LAST,
                                 preferred_element_type=jnp.float32)
        s = jnp.maximum(base + l0, 0.0) + jnp.maximum(base + l1, 0.0)
        s16_scr[:, sl] = s.astype(jnp.bfloat16)
    s16 = s16_scr[...]

    third = D_MODEL // 3
    for q in range(3):
        lo, hi = q * third, (q + 1) * third
        c_q = jax.lax.dot_general(s16, wo_ref[lo:hi, :].astype(jnp.bfloat16),
                                  _CONTRACT_LAST,
                                  preferred_element_type=jnp.float32)
        out_ref[:, lo:hi] = jnp.where(ff == 0, c_q, out_ref[:, lo:hi] + c_q)


def kernel(hidden_states, top_k_indices, top_k_values, Wi, Wo, lora_As, lora_Bs):
    batch, seq, d_model = hidden_states.shape
    rows = batch * seq
    x = hidden_states.reshape(rows, d_model)
    idx = top_k_indices.astype(jnp.int32)
    vals = top_k_values.astype(jnp.float32)

    out = pl.pallas_call(
        _moe_kernel,
        grid_spec=pltpu.PrefetchScalarGridSpec(
            num_scalar_prefetch=2,
            grid=(N_FF,),
            in_specs=[
                pl.BlockSpec((rows, d_model), lambda ff, idx, val: (0, 0)),
                pl.BlockSpec((FF_TILE, d_model), lambda ff, idx, val: (ff, 0)),
                pl.BlockSpec((d_model, FF_TILE), lambda ff, idx, val: (0, ff)),
                pl.BlockSpec((1, RANK, d_model), lambda ff, idx, val: (idx[0], 0, 0)),
                pl.BlockSpec((1, RANK, d_model), lambda ff, idx, val: (idx[1], 0, 0)),
                pl.BlockSpec((1, FF_TILE, RANK), lambda ff, idx, val: (idx[0], ff, 0)),
                pl.BlockSpec((1, FF_TILE, RANK), lambda ff, idx, val: (idx[1], ff, 0)),
            ],
            out_specs=pl.BlockSpec((rows, d_model), lambda ff, idx, val: (0, 0)),
            scratch_shapes=[
                pltpu.VMEM((rows, RANK), jnp.bfloat16),
                pltpu.VMEM((rows, RANK), jnp.bfloat16),
                pltpu.VMEM((rows, d_model), jnp.bfloat16),
                pltpu.VMEM((rows, FF_TILE), jnp.bfloat16),
            ],
        ),
        out_shape=jax.ShapeDtypeStruct((rows, d_model), jnp.float32),
    )(idx, vals, x, Wi, Wo, lora_As, lora_As, lora_Bs, lora_Bs)
    return out.reshape(batch, seq, d_model)
